# baseline (speedup 1.0000x reference)
import functools

import jax
import jax.numpy as jnp
from jax import lax
from jax.experimental import pallas as pl
from jax.experimental.pallas import tpu as pltpu
from jax.experimental.pallas import tpu_sc as plsc

_N = 10000
_E = 320000
_D = 128
_DE = 16
_H = 128
_NG = 64
_NC = 3

_NCORES = 2
_NSUB = 16
_NW = _NCORES * _NSUB
_EPT = _E // _NW
_CH = 40
_NCHUNK = _EPT // _CH
_NPAD = 10240
_RPT = _NPAD // _NSUB

_LOG1P = (
    2.21597649e-07, 9.99970243e-01, -4.99333949e-01, 3.27511714e-01,
    -2.23966899e-01, 1.31989662e-01, -5.32674777e-02, 1.02438286e-02,
)


def _log1p_poly(u):
    p = jnp.full_like(u, _LOG1P[7])
    for c in range(6, -1, -1):
        p = p * u + _LOG1P[c]
    return p


def _embed_proj_body(xr, wer, ber, wdr, wsr, hr, tdr, tsr):
    h = jnp.dot(xr[...], wer[...], preferred_element_type=jnp.float32) + ber[...]
    hr[...] = h
    tdr[...] = jnp.dot(h, wdr[...], preferred_element_type=jnp.float32)
    tsr[...] = jnp.dot(h, wsr[...], preferred_element_type=jnp.float32)


def _embed_proj(x, w_emb, b_emb, wd, ws):
    return pl.pallas_call(
        _embed_proj_body,
        out_shape=(
            jax.ShapeDtypeStruct((_N, _H), jnp.float32),
            jax.ShapeDtypeStruct((_N, 2 * _H), jnp.float32),
            jax.ShapeDtypeStruct((_N, 2 * _H), jnp.float32),
        ),
    )(x, w_emb, b_emb.reshape(1, _H), wd, ws)


def _update_proj_body(hr, pr, wdr, wsr, hor, tdr, tsr):
    h = jax.nn.softplus(hr[...] + pr[0] + pr[1])
    hor[...] = h
    tdr[...] = jnp.dot(h, wdr[...], preferred_element_type=jnp.float32)
    tsr[...] = jnp.dot(h, wsr[...], preferred_element_type=jnp.float32)


def _update_proj(h, parts, wd, ws):
    return pl.pallas_call(
        _update_proj_body,
        out_shape=(
            jax.ShapeDtypeStruct((_N, _H), jnp.float32),
            jax.ShapeDtypeStruct((_N, 2 * _H), jnp.float32),
            jax.ShapeDtypeStruct((_N, 2 * _H), jnp.float32),
        ),
    )(h, parts, wd, ws)


def _eproj_body(er, wr, br, outr):
    outr[...] = jnp.dot(er[...], wr[...], preferred_element_type=jnp.float32) + br[...]


def _eproj(edge_attr, we, be):
    blk = 8000
    return pl.pallas_call(
        _eproj_body,
        grid=(_E // blk,),
        in_specs=[
            pl.BlockSpec((blk, _DE), lambda i: (i, 0)),
            pl.BlockSpec((_DE, 2 * _H), lambda i: (0, 0)),
            pl.BlockSpec((1, 2 * _H), lambda i: (0, 0)),
        ],
        out_specs=pl.BlockSpec((blk, 2 * _H), lambda i: (i, 0)),
        out_shape=jax.ShapeDtypeStruct((_E, 2 * _H), jnp.float32),
    )(edge_attr, we, be.reshape(1, 2 * _H))


def _update_pool_body(hr, pr, br, w1r, b1r, w2r, b2r, outr):
    h = jax.nn.softplus(hr[...] + pr[0] + pr[1])
    seg = lax.broadcasted_iota(jnp.int32, (_NG, _N), 0)
    onehot = (seg == br[...]).astype(jnp.float32)
    sums = jnp.dot(onehot, h, preferred_element_type=jnp.float32)
    counts = jnp.sum(onehot, axis=1, keepdims=True)
    pooled = sums / jnp.maximum(counts, 1.0)
    r = jax.nn.softplus(
        jnp.dot(pooled, w1r[...], preferred_element_type=jnp.float32) + b1r[...])
    outr[...] = jnp.dot(r, w2r[...], preferred_element_type=jnp.float32) + b2r[...]


def _update_pool(h, parts, batch_row, w1, b1, w2, b2):
    return pl.pallas_call(
        _update_pool_body,
        out_shape=jax.ShapeDtypeStruct((_NG, 1), jnp.float32),
    )(h, parts, batch_row, w1, b1.reshape(1, _H), w2, b2.reshape(1, 1))


_mesh = plsc.VectorSubcoreMesh(
    core_axis_name="c", subcore_axis_name="s",
    num_cores=_NCORES, num_subcores=_NSUB)


@functools.partial(
    pl.kernel,
    out_type=jax.ShapeDtypeStruct((_NCORES, _NPAD, _H), jnp.float32),
    mesh=_mesh,
    scratch_types=[
        pltpu.VMEM((_CH,), jnp.int32),
        pltpu.VMEM((_CH,), jnp.int32),
        pltpu.VMEM((_CH, 2 * _H), jnp.float32),
        pltpu.VMEM((_CH, 2 * _H), jnp.float32),
        pltpu.VMEM((_CH, 2 * _H), jnp.float32),
        pltpu.VMEM((_CH, _H), jnp.float32),
        pltpu.VMEM_SHARED((_NPAD, _H), jnp.float32),
        pltpu.SemaphoreType.DMA,
        pltpu.SemaphoreType.DMA,
        pltpu.SemaphoreType.DMA,
    ],
)
def _msg(tdst_hbm, tsrc_hbm, cmg_hbm, dst_hbm, src_hbm, out_hbm,
         idx_d, idx_s, gd, gs, cc, m, accum, sem_d, sem_s, sem_c):
    c = lax.axis_index("c")
    s = lax.axis_index("s")
    row0 = s * _RPT

    zv = jnp.zeros((16,), jnp.float32)

    def zrow(i, _):
        for k in range(_H // 16):
            m[i, pl.ds(k * 16, 16)] = zv
        return 0

    lax.fori_loop(0, _CH, zrow, 0)
    for k in range(_RPT // _CH):
        pltpu.sync_copy(m, accum.at[pl.ds(row0 + k * _CH, _CH)])
    plsc.subcore_barrier()

    base = (c * _NSUB + s) * _EPT

    def chunk(i, _):
        off = base + i * _CH
        pltpu.sync_copy(dst_hbm.at[pl.ds(off, _CH)], idx_d)
        pltpu.sync_copy(src_hbm.at[pl.ds(off, _CH)], idx_s)
        cp_d = pltpu.async_copy(tdst_hbm.at[idx_d], gd, sem_d)
        cp_s = pltpu.async_copy(tsrc_hbm.at[idx_s], gs, sem_s)
        cp_c = pltpu.async_copy(cmg_hbm.at[pl.ds(off, _CH)], cc, sem_c)
        cp_d.wait()
        cp_s.wait()
        cp_c.wait()

        def edge(j, _):
            for k in range(_H // 16):
                lo = pl.ds(k * 16, 16)
                hi = pl.ds(_H + k * 16, 16)
                sv = gd[j, lo] + gs[j, lo] + cc[j, lo]
                tv = gd[j, hi] + gs[j, hi] + cc[j, hi]
                sig = 1.0 / (1.0 + jnp.exp(-sv))
                sp = jnp.maximum(tv, 0.0) + _log1p_poly(jnp.exp(-jnp.abs(tv)))
                m[j, lo] = sig * sp
            return 0

        lax.fori_loop(0, _CH, edge, 0)
        pltpu.sync_copy(m, accum.at[idx_d], add=True)
        return 0

    lax.fori_loop(0, _NCHUNK, chunk, 0)
    plsc.subcore_barrier()

    def wrow(k, _):
        rows = pl.ds(row0 + k * _CH, _CH)
        pltpu.sync_copy(accum.at[rows], m)
        pltpu.sync_copy(m, out_hbm.at[c, rows])
        return 0

    lax.fori_loop(0, _RPT // _CH, wrow, 0)


def kernel(x, edge_index, edge_attr, batch, W_emb, b_emb, Wm, bm, Wg, bg,
           W_r1, b_r1, W_r2, b_r2):
    src = edge_index[0]
    dst = edge_index[1]

    wd = [jnp.concatenate([Wm[c][:_H], Wg[c][:_H]], axis=1) for c in range(_NC)]
    ws = [jnp.concatenate([Wm[c][_H:2 * _H], Wg[c][_H:2 * _H]], axis=1)
          for c in range(_NC)]
    we = [jnp.concatenate([Wm[c][2 * _H:], Wg[c][2 * _H:]], axis=1)
          for c in range(_NC)]
    be = [jnp.concatenate([bm[c], bg[c]]) for c in range(_NC)]

    cmg = [_eproj(edge_attr, we[c], be[c]) for c in range(_NC)]

    h, td, ts = _embed_proj(x, W_emb, b_emb, wd[0], ws[0])
    out = None
    for c in range(_NC):
        parts = _msg(td, ts, cmg[c], dst, src)[:, :_N]
        if c < _NC - 1:
            h, td, ts = _update_proj(h, parts, wd[c + 1], ws[c + 1])
        else:
            out = _update_pool(h, parts, batch.reshape(1, _N),
                               W_r1, b_r1, W_r2, b_r2)
    return out


# speedup vs baseline: 1.1742x; 1.1742x over previous
import functools

import jax
import jax.numpy as jnp
from jax import lax
from jax.experimental import pallas as pl
from jax.experimental.pallas import tpu as pltpu
from jax.experimental.pallas import tpu_sc as plsc

_N = 10000
_E = 320000
_D = 128
_DE = 16
_H = 128
_NG = 64
_NC = 3

_NCORES = 2
_NSUB = 16
_NW = _NCORES * _NSUB
_EPT = _E // _NW
_CH = 16
_NCHUNK = _EPT // _CH
_PH = 125
_NPH = _NCHUNK // _PH
_EPP = _PH * _CH
_NPAD = 10240
_RPT = _NPAD // _NSUB

_LOG1P = (
    2.21597649e-07, 9.99970243e-01, -4.99333949e-01, 3.27511714e-01,
    -2.23966899e-01, 1.31989662e-01, -5.32674777e-02, 1.02438286e-02,
)


def _log1p_poly(u):
    p = u * _LOG1P[7] + _LOG1P[6]
    for c in range(5, -1, -1):
        p = p * u + _LOG1P[c]
    return p


def _embed_proj_body(xr, wer, ber, wdr, wsr, hr, tdr, tsr):
    h = jnp.dot(xr[...], wer[...], preferred_element_type=jnp.float32) + ber[...]
    hr[...] = h
    tdr[...] = jnp.dot(h, wdr[...], preferred_element_type=jnp.float32)
    tsr[...] = jnp.dot(h, wsr[...], preferred_element_type=jnp.float32)


def _embed_proj(x, w_emb, b_emb, wd, ws):
    return pl.pallas_call(
        _embed_proj_body,
        out_shape=(
            jax.ShapeDtypeStruct((_N, _H), jnp.float32),
            jax.ShapeDtypeStruct((_N, 2 * _H), jnp.float32),
            jax.ShapeDtypeStruct((_N, 2 * _H), jnp.float32),
        ),
    )(x, w_emb, b_emb.reshape(1, _H), wd, ws)


def _update_proj_body(hr, pr, wdr, wsr, hr_o, tdr, tsr):
    h = jax.nn.softplus(hr[...] + pr[0] + pr[1])
    hr_o[...] = h
    tdr[...] = jnp.dot(h, wdr[...], preferred_element_type=jnp.float32)
    tsr[...] = jnp.dot(h, wsr[...], preferred_element_type=jnp.float32)


def _update_proj(h, parts, wd, ws):
    return pl.pallas_call(
        _update_proj_body,
        out_shape=(
            jax.ShapeDtypeStruct((_N, _H), jnp.float32),
            jax.ShapeDtypeStruct((_N, 2 * _H), jnp.float32),
            jax.ShapeDtypeStruct((_N, 2 * _H), jnp.float32),
        ),
    )(h, parts, wd, ws)


def _eproj_body(er, wr, br, outr):
    outr[...] = jnp.dot(er[...], wr[...], preferred_element_type=jnp.float32) + br[...]


def _eproj(edge_attr, we, be):
    blk = 8000
    return pl.pallas_call(
        _eproj_body,
        grid=(_E // blk,),
        in_specs=[
            pl.BlockSpec((blk, _DE), lambda i: (i, 0)),
            pl.BlockSpec((_DE, 2 * _H), lambda i: (0, 0)),
            pl.BlockSpec((1, 2 * _H), lambda i: (0, 0)),
        ],
        out_specs=pl.BlockSpec((blk, 2 * _H), lambda i: (i, 0)),
        out_shape=jax.ShapeDtypeStruct((_E, 2 * _H), jnp.float32),
    )(edge_attr, we, be.reshape(1, 2 * _H))


def _update_pool_body(hr, pr, br, w1r, b1r, w2r, b2r, outr):
    h = jax.nn.softplus(hr[...] + pr[0] + pr[1])
    seg = lax.broadcasted_iota(jnp.int32, (_NG, _N), 0)
    onehot = (seg == br[...]).astype(jnp.float32)
    sums = jnp.dot(onehot, h, preferred_element_type=jnp.float32)
    counts = jnp.sum(onehot, axis=1, keepdims=True)
    pooled = sums / jnp.maximum(counts, 1.0)
    r = jax.nn.softplus(
        jnp.dot(pooled, w1r[...], preferred_element_type=jnp.float32) + b1r[...])
    outr[...] = jnp.dot(r, w2r[...], preferred_element_type=jnp.float32) + b2r[...]


def _update_pool(h, parts, batch_row, w1, b1, w2, b2):
    return pl.pallas_call(
        _update_pool_body,
        out_shape=jax.ShapeDtypeStruct((_NG, 1), jnp.float32),
    )(h, parts, batch_row, w1, b1.reshape(1, _H), w2, b2.reshape(1, 1))


_mesh = plsc.VectorSubcoreMesh(
    core_axis_name="c", subcore_axis_name="s",
    num_cores=_NCORES, num_subcores=_NSUB)


@functools.partial(
    pl.kernel,
    out_type=jax.ShapeDtypeStruct((_NCORES, _NPAD, _H), jnp.float32),
    mesh=_mesh,
    scratch_types=[
        pltpu.VMEM((_EPP,), jnp.int32),
        pltpu.VMEM((_EPP,), jnp.int32),
        pltpu.VMEM((_CH, 2 * _H), jnp.float32),
        pltpu.VMEM((_CH, 2 * _H), jnp.float32),
        pltpu.VMEM((_CH, 2 * _H), jnp.float32),
        pltpu.VMEM((_CH, 2 * _H), jnp.float32),
        pltpu.VMEM((_CH, 2 * _H), jnp.float32),
        pltpu.VMEM((_CH, 2 * _H), jnp.float32),
        pltpu.VMEM((_CH, _H), jnp.float32),
        pltpu.VMEM((_CH, _H), jnp.float32),
        pltpu.VMEM_SHARED((_NPAD, _H), jnp.float32),
        pltpu.SemaphoreType.DMA,
        pltpu.SemaphoreType.DMA,
        pltpu.SemaphoreType.DMA,
        pltpu.SemaphoreType.DMA,
        pltpu.SemaphoreType.DMA,
        pltpu.SemaphoreType.DMA,
    ],
)
def _msg(td_hbm, ts_hbm, cmg_hbm, dst_hbm, src_hbm, out_hbm,
         idxd, idxs, gd0, gd1, gs0, gs1, cc0, cc1, m0, m1, accum,
         sgd0, sgd1, sgs0, sgs1, scc0, scc1):
    c = lax.axis_index("c")
    s = lax.axis_index("s")
    row0 = s * _RPT
    gd = (gd0, gd1)
    gs = (gs0, gs1)
    cc = (cc0, cc1)
    mm = (m0, m1)
    sgd = (sgd0, sgd1)
    sgs = (sgs0, sgs1)
    scc = (scc0, scc1)
    tile = c * _NSUB + s
    ebase = tile * _EPT
    zv = jnp.zeros((16,), jnp.float32)

    def zrow(i, _):
        for k in range(_H // 16):
            m0[i, pl.ds(k * 16, 16)] = zv
        return 0

    lax.fori_loop(0, _CH, zrow, 0)

    def zcopy(k, _):
        pltpu.sync_copy(m0, accum.at[pl.ds(row0 + k * _CH, _CH)])
        return 0

    lax.fori_loop(0, _RPT // _CH, zcopy, 0)
    plsc.subcore_barrier()

    def issue(kk, p, b):
        ivd = idxd[pl.ds(kk * _CH, _CH)]
        ivs = idxs[pl.ds(kk * _CH, _CH)]
        eoff = ebase + p * _EPP + kk * _CH
        pltpu.async_copy(td_hbm.at[ivd], gd[b], sgd[b])
        pltpu.async_copy(ts_hbm.at[ivs], gs[b], sgs[b])
        pltpu.async_copy(cmg_hbm.at[pl.ds(eoff, _CH)], cc[b], scc[b])

    def wait(kk, b):
        ivd = idxd[pl.ds(kk * _CH, _CH)]
        ivs = idxs[pl.ds(kk * _CH, _CH)]
        pltpu.make_async_copy(td_hbm.at[ivd], gd[b], sgd[b]).wait()
        pltpu.make_async_copy(ts_hbm.at[ivs], gs[b], sgs[b]).wait()
        pltpu.make_async_copy(cmg_hbm.at[pl.ds(0, _CH)], cc[b], scc[b]).wait()

    def compute(b):
        g_d, g_s, c_c, m = gd[b], gs[b], cc[b], mm[b]

        def edge(j, _):
            for k in range(_H // 16):
                lo = pl.ds(k * 16, 16)
                hi = pl.ds(_H + k * 16, 16)
                sv = g_d[j, lo] + g_s[j, lo] + c_c[j, lo]
                tv = g_d[j, hi] + g_s[j, hi] + c_c[j, hi]
                sig = 1.0 / (1.0 + jnp.exp(-sv))
                sp = jnp.maximum(tv, 0.0) + _log1p_poly(jnp.exp(-jnp.abs(tv)))
                m[j, lo] = sig * sp
            return 0

        lax.fori_loop(0, _CH, edge, 0)

    def step_full(kk, p, b):
        wait(kk, b)
        compute(b)
        pltpu.sync_copy(mm[b], accum.at[idxd[pl.ds(kk * _CH, _CH)]], add=True)
        issue(kk + 2, p, b)

    def step_end(kk, b):
        wait(kk, b)
        compute(b)
        pltpu.sync_copy(mm[b], accum.at[idxd[pl.ds(kk * _CH, _CH)]], add=True)

    def phase(p, _):
        off = ebase + p * _EPP
        pltpu.sync_copy(dst_hbm.at[pl.ds(off, _EPP)], idxd)
        pltpu.sync_copy(src_hbm.at[pl.ds(off, _EPP)], idxs)
        issue(0, p, 0)
        issue(1, p, 1)

        def qbody(q, _):
            step_full(2 * q, p, 0)
            step_full(2 * q + 1, p, 1)
            return 0

        lax.fori_loop(0, (_PH - 3) // 2, qbody, 0)
        step_full(_PH - 3, p, 0)
        step_end(_PH - 2, 1)
        step_end(_PH - 1, 0)
        return 0

    lax.fori_loop(0, _NPH, phase, 0)
    plsc.subcore_barrier()

    def wrow(k, _):
        rows = pl.ds(row0 + k * _CH, _CH)
        pltpu.sync_copy(accum.at[rows], m0)
        pltpu.sync_copy(m0, out_hbm.at[c, rows])
        return 0

    lax.fori_loop(0, _RPT // _CH, wrow, 0)


def kernel(x, edge_index, edge_attr, batch, W_emb, b_emb, Wm, bm, Wg, bg,
           W_r1, b_r1, W_r2, b_r2):
    src = edge_index[0]
    dst = edge_index[1]

    wd = [jnp.concatenate([Wm[c][:_H], Wg[c][:_H]], axis=1) for c in range(_NC)]
    ws = [jnp.concatenate([Wm[c][_H:2 * _H], Wg[c][_H:2 * _H]], axis=1)
          for c in range(_NC)]
    we = [jnp.concatenate([Wm[c][2 * _H:], Wg[c][2 * _H:]], axis=1)
          for c in range(_NC)]
    be = [jnp.concatenate([bm[c], bg[c]]) for c in range(_NC)]

    cmg = [_eproj(edge_attr, we[c], be[c]) for c in range(_NC)]

    h, td, ts = _embed_proj(x, W_emb, b_emb, wd[0], ws[0])
    out = None
    for c in range(_NC):
        parts = _msg(td, ts, cmg[c], dst, src)[:, :_N]
        if c < _NC - 1:
            h, td, ts = _update_proj(h, parts, wd[c + 1], ws[c + 1])
        else:
            out = _update_pool(h, parts, batch.reshape(1, _N),
                               W_r1, b_r1, W_r2, b_r2)
    return out
